# baseline (speedup 1.0000x reference)
import functools
import jax
import jax.numpy as jnp
from jax import lax
from jax.experimental import pallas as pl
from jax.experimental.pallas import tpu as pltpu
from jax.experimental.pallas import tpu_sc as plsc

_NEG_RATIO = 3
_P = 8732
_P4 = 8736
_NCH = _P4 // 16
_PAD_KEY = -2139095041


def _tc_body(conf_ref, locst_ref, gtt_ref, labr_ref,
             key_ref, ce_ref, stat_ref, sum_ref, acc_ref):
    b = pl.program_id(0)
    nb = pl.num_programs(0)
    P, C = conf_ref.shape[1], conf_ref.shape[2]

    @pl.when(b == 0)
    def _():
        acc_ref[0] = 0.0
        acc_ref[1] = 0.0
        acc_ref[2] = 0.0
        acc_ref[3] = 0.0

    xt = jnp.swapaxes(conf_ref[0], 0, 1)
    lab = labr_ref[0]
    colmax = jnp.max(xt, axis=0, keepdims=True)
    s = jnp.sum(jnp.exp(xt - colmax), axis=0, keepdims=True)
    lse = colmax + jnp.log(s)
    ci = jax.lax.broadcasted_iota(jnp.int32, (C, P), 0)
    x_at_lab = jnp.sum(jnp.where(ci == lab, xt, 0.0), axis=0, keepdims=True)
    ce = lse - x_at_lab
    pos = lab > 0
    npos = jnp.sum(pos.astype(jnp.int32))

    loss = lse - xt[0:1, :]
    loss_hn = jnp.where(pos, jnp.float32(-jnp.inf), loss)
    bits = jax.lax.bitcast_convert_type(loss_hn, jnp.int32)
    key = jnp.where(bits >= 0, bits, bits ^ jnp.int32(0x7FFFFFFF))

    ce_neg = jnp.where(pos, 0.0, ce)
    cepos = jnp.sum(jnp.where(pos, ce, 0.0))

    pad_i = jnp.full((1, _P4 - P), jnp.int32(_PAD_KEY), jnp.int32)
    pad_f = jnp.zeros((1, _P4 - P), jnp.float32)
    key_ref[0] = jnp.concatenate([key, pad_i], axis=1)
    ce_ref[0] = jnp.concatenate([ce_neg, pad_f], axis=1)

    dl = locst_ref[0] - gtt_ref[0]
    ad = jnp.abs(dl)
    e = jnp.where(ad < 1.0, 0.5 * dl * dl, ad - 0.5)
    rs = jnp.sum(e, axis=0, keepdims=True)
    sl1 = jnp.sum(jnp.where(pos, rs, 0.0))

    npos_f = npos.astype(jnp.float32)
    stat_ref[0] = jnp.zeros((1, 16), jnp.float32) + npos_f

    acc_ref[0] += sl1
    acc_ref[1] += npos_f
    acc_ref[2] += jnp.where(npos > 0, 0.0, 3.0)
    acc_ref[3] += cepos

    @pl.when(b == nb - 1)
    def _():
        tp = acc_ref[1]
        l1 = acc_ref[0] / jnp.maximum(tp, 1.0)
        den = jnp.maximum(tp + acc_ref[2], 1.0)
        li = jax.lax.broadcasted_iota(jnp.int32, (1, 128), 1)
        sum_ref[...] = jnp.where(
            li == 0, l1, jnp.where(li == 1, den,
                                   jnp.where(li == 2, acc_ref[3], 0.0)))


_GDN = lax.GatherDimensionNumbers(
    offset_dims=(), collapsed_slice_dims=(0,), start_index_map=(0,))


def _splat(acc):
    for sh in (8, 4, 2, 1):
        idx = (lax.iota(jnp.int32, 16) + sh) % 16
        g = lax.gather(acc, idx[:, None], _GDN, (1,),
                       mode=lax.GatherScatterMode.PROMISE_IN_BOUNDS)
        acc = acc + g
    return acc


def _sc_body(key_hbm, ce_hbm, st_hbm, idx_hbm, out_hbm,
             key_v, ce_v, st_v, idx_v, res_v, tr_v, t_v, j_v):
    c = lax.axis_index("c")
    s = lax.axis_index("s")
    wid = s * 2 + c
    pltpu.sync_copy(key_hbm.at[wid], key_v)
    pltpu.sync_copy(ce_hbm.at[wid], ce_v)
    pltpu.sync_copy(st_hbm.at[wid], st_v)
    pltpu.sync_copy(idx_hbm, idx_v)
    npos = st_v[...].astype(jnp.int32)
    K = jnp.where(npos > 0, _NEG_RATIO * npos,
                  jnp.full((16,), _NEG_RATIO, jnp.int32))
    fast = 4 * npos[0] >= _P

    zi = jnp.zeros((16,), jnp.int32)
    zf = jnp.zeros((16,), jnp.float32)
    one = jnp.full((16,), 1, jnp.int32)

    @pl.when(fast)
    def _():
        def cstep(i, acc):
            return acc + ce_v[pl.ds(i * 16, 16)]
        res_v[...] = lax.fori_loop(0, _NCH, cstep, zf)

    @pl.when(jnp.logical_not(fast))
    def _():
        int_min = jnp.int32(-2147483648)
        t = jnp.full((16,), int_min, jnp.int32)

        def count_ge(i, cnt):
            mk = key_v[pl.ds(i * 16, 16)] >= tr_v[...]
            return cnt + jnp.where(mk, one, zi)

        for sbi in range(32):
            sb = 31 - sbi
            if sb == 31:
                trial = t ^ int_min
            else:
                trial = t | jnp.int32(1 << sb)
            tr_v[...] = trial
            cnt = _splat(lax.fori_loop(0, _NCH, count_ge, zi))
            t = jnp.where(cnt >= K, trial, t)

        t_v[...] = t

        def gstep(i, carry):
            cg, ssum = carry
            kk = key_v[pl.ds(i * 16, 16)]
            g = kk > t_v[...]
            cg = cg + jnp.where(g, one, zi)
            ssum = ssum + jnp.where(g, ce_v[pl.ds(i * 16, 16)], 0.0)
            return (cg, ssum)

        cnt_gt, sum_gt = lax.fori_loop(0, _NCH, gstep, (zi, zf))
        m = K - _splat(cnt_gt)

        def count_eq_below(i, cnt):
            kk = key_v[pl.ds(i * 16, 16)]
            idx = idx_v[pl.ds(i * 16, 16)]
            inner = jnp.where(idx < tr_v[...], one, zi)
            return cnt + jnp.where(kk == t_v[...], inner, zi)

        j = zi
        for jbi in range(14):
            trial = j | jnp.int32(1 << (13 - jbi))
            tr_v[...] = trial
            c2 = _splat(lax.fori_loop(0, _NCH, count_eq_below, zi))
            j = jnp.where(c2 < m, trial, j)

        j_v[...] = j

        def estep(i, ssum):
            kk = key_v[pl.ds(i * 16, 16)]
            idx = idx_v[pl.ds(i * 16, 16)]
            ce = ce_v[pl.ds(i * 16, 16)]
            inner = jnp.where(idx <= j_v[...], ce, 0.0)
            return ssum + jnp.where(kk == t_v[...], inner, 0.0)

        res_v[...] = sum_gt + lax.fori_loop(0, _NCH, estep, zf)

    pltpu.sync_copy(res_v, out_hbm.at[wid])


def kernel(confidence, predicted_locations, labels, gt_locations):
    B, P, C = confidence.shape
    locs_t = jnp.transpose(predicted_locations, (0, 2, 1))
    gt_t = jnp.transpose(gt_locations, (0, 2, 1))
    lab_row = labels[:, None, :]

    key_r, ce_r, stat_r, sum_r = pl.pallas_call(
        _tc_body,
        grid=(B,),
        in_specs=[
            pl.BlockSpec((1, P, C), lambda b: (b, 0, 0)),
            pl.BlockSpec((1, 4, P), lambda b: (b, 0, 0)),
            pl.BlockSpec((1, 4, P), lambda b: (b, 0, 0)),
            pl.BlockSpec((1, 1, P), lambda b: (b, 0, 0)),
        ],
        out_specs=[
            pl.BlockSpec((1, 1, _P4), lambda b: (b, 0, 0)),
            pl.BlockSpec((1, 1, _P4), lambda b: (b, 0, 0)),
            pl.BlockSpec((1, 1, 16), lambda b: (b, 0, 0)),
            pl.BlockSpec((1, 128), lambda b: (0, 0)),
        ],
        out_shape=[
            jax.ShapeDtypeStruct((B, 1, _P4), jnp.int32),
            jax.ShapeDtypeStruct((B, 1, _P4), jnp.float32),
            jax.ShapeDtypeStruct((B, 1, 16), jnp.float32),
            jax.ShapeDtypeStruct((1, 128), jnp.float32),
        ],
        scratch_shapes=[pltpu.SMEM((8,), jnp.float32)],
    )(confidence, locs_t, gt_t, lab_row)

    sc_select = functools.partial(
        pl.kernel,
        out_type=jax.ShapeDtypeStruct((B, 16), jnp.float32),
        mesh=plsc.VectorSubcoreMesh(core_axis_name="c", subcore_axis_name="s"),
        scratch_types=[
            pltpu.VMEM((_P4,), jnp.int32),
            pltpu.VMEM((_P4,), jnp.float32),
            pltpu.VMEM((16,), jnp.float32),
            pltpu.VMEM((_P4,), jnp.int32),
            pltpu.VMEM((16,), jnp.float32),
            pltpu.VMEM((16,), jnp.int32),
            pltpu.VMEM((16,), jnp.int32),
            pltpu.VMEM((16,), jnp.int32),
        ],
    )(_sc_body)

    prior_idx = jnp.arange(_P4, dtype=jnp.int32)
    cls_rows = sc_select(key_r.reshape(B, _P4), ce_r.reshape(B, _P4),
                         stat_r.reshape(B, 16), prior_idx)

    smooth_l1_loss = sum_r[0, 0]
    classification_loss = (sum_r[0, 2] + jnp.sum(cls_rows)) / sum_r[0, 1]
    return (smooth_l1_loss, classification_loss)
